# Initial kernel scaffold; baseline (speedup 1.0000x reference)
import jax
import jax.numpy as jnp
from jax.experimental import pallas as pl


def kernel(x, token_table, pos_table):
    raise NotImplementedError("write your pallas kernel here")


# speedup vs baseline: 1.2677x; 1.2677x over previous
import functools

import jax
import jax.numpy as jnp
from jax import lax
from jax.experimental import pallas as pl
from jax.experimental.pallas import tpu as pltpu
from jax.experimental.pallas import tpu_sc as plsc

_B, _L, _D = 4096, 200, 32
_N = _B * _L
_CH = 128


def _make_kernel():
    mesh = plsc.VectorSubcoreMesh(core_axis_name="c", subcore_axis_name="s")
    nc, ns = mesh.num_cores, mesh.num_subcores
    nw = nc * ns
    rows_w = _N // nw
    g_steps = rows_w // _CH
    assert _N % (nw * _CH) == 0

    @functools.partial(
        pl.kernel,
        out_type=jax.ShapeDtypeStruct((_N, _D), jnp.float32),
        mesh=mesh,
        compiler_params=pltpu.CompilerParams(use_tc_tiling_on_sc=False),
        scratch_types=[
            pltpu.VMEM((g_steps, _CH), jnp.int32),
            pltpu.VMEM((2 * _L, _D), jnp.float32),
            pltpu.VMEM((_CH, _D), jnp.float32),
            pltpu.VMEM((_CH, _D), jnp.float32),
            pltpu.SemaphoreType.DMA,
            pltpu.SemaphoreType.DMA,
            pltpu.SemaphoreType.DMA,
            pltpu.SemaphoreType.DMA,
        ],
    )
    def emb_kernel(tok_hbm, xidx_hbm, pos_hbm, out_hbm,
                   idx_v, pos2_v, dest0, dest1, gsem0, gsem1, ssem0, ssem1):
        wid = lax.axis_index("s") * nc + lax.axis_index("c")
        row0 = wid * rows_w

        pltpu.sync_copy(xidx_hbm.at[wid], idx_v)
        pltpu.sync_copy(pos_hbm, pos2_v.at[pl.ds(0, _L)])
        pltpu.sync_copy(pos_hbm, pos2_v.at[pl.ds(_L, _L)])

        pltpu.async_copy(tok_hbm.at[idx_v.at[0]], dest0, gsem0)

        def step(g, dest_b, gsem_b, ssem_b, dest_n, gsem_n, ssem_n):
            rowbase = row0 + g * _CH

            @pl.when(g >= 1)
            def _():
                pltpu.make_async_copy(
                    dest_n, out_hbm.at[pl.ds(rowbase - _CH, _CH)], ssem_n
                ).wait()

            @pl.when(g + 1 < g_steps)
            def _():
                pltpu.async_copy(tok_hbm.at[idx_v.at[g + 1]], dest_n, gsem_n)

            pltpu.make_async_copy(
                tok_hbm.at[idx_v.at[g]], dest_b, gsem_b
            ).wait()

            p0 = lax.rem(g * _CH, _L)

            def add_rows(i, carry):
                r = i * 8
                for k in range(8):
                    pr = p0 + r + k
                    plsc.addupdate(dest_b.at[r + k, pl.ds(0, 16)],
                                   pos2_v[pr, pl.ds(0, 16)])
                    plsc.addupdate(dest_b.at[r + k, pl.ds(16, 16)],
                                   pos2_v[pr, pl.ds(16, 16)])
                return carry

            lax.fori_loop(0, _CH // 8, add_rows, 0)

            pltpu.async_copy(dest_b, out_hbm.at[pl.ds(rowbase, _CH)], ssem_b)

        def outer(i, carry):
            g = i * 2
            step(g, dest0, gsem0, ssem0, dest1, gsem1, ssem1)
            step(g + 1, dest1, gsem1, ssem1, dest0, gsem0, ssem0)
            return carry

        lax.fori_loop(0, g_steps // 2, outer, 0)

        pltpu.make_async_copy(
            dest1, out_hbm.at[pl.ds(row0 + (g_steps - 1) * _CH, _CH)], ssem1
        ).wait()

    return emb_kernel, nw, g_steps


def kernel(x, token_table, pos_table):
    emb, nw, g_steps = _make_kernel()
    xr = x.astype(jnp.int32).reshape(nw, g_steps, _CH)
    out = emb(token_table, xr, pos_table)
    return out.reshape(_B, _L, _D)
